# baseline (speedup 1.0000x reference)
import functools

import jax
import jax.numpy as jnp
from jax import lax
from jax.experimental import pallas as pl
from jax.experimental.pallas import tpu as pltpu
from jax.experimental.pallas import tpu_sc as plsc

EPS = 1e-12
_NC = 2
_NS = 16

_GDN = lax.GatherDimensionNumbers(
    offset_dims=(), collapsed_slice_dims=(0,), start_index_map=(0,))


def _shuffle(x, perm):
  return lax.gather(x, perm[:, None], _GDN, (1,),
                    mode=lax.GatherScatterMode.PROMISE_IN_BOUNDS)


def _lane_allsum(x, perms):
  for p in perms:
    x = x + _shuffle(x, p)
  return x


def _sc_body(L, D, rows_per_w, word_hbm, pos_hbm, cat_hbm, gamma_hbm,
             beta_hbm, tag_hbm, catid_hbm, out_hbm,
             idx_v, pos_v, catid_v, catrows_v, gam_v, bet_v, rows_v, sem):
  tok_per_w = rows_per_w * L
  wid = lax.axis_index("c") * _NS + lax.axis_index("s")
  row0 = wid * rows_per_w
  tok0 = wid * tok_per_w

  pltpu.sync_copy(tag_hbm.at[pl.ds(tok0, tok_per_w)], idx_v)
  pltpu.sync_copy(pos_hbm.at[pl.ds(0, L)], pos_v)
  pltpu.sync_copy(catid_hbm.at[pl.ds(row0, rows_per_w)], catid_v)
  pltpu.async_copy(cat_hbm.at[catid_v], catrows_v, sem).wait()
  pltpu.sync_copy(gamma_hbm, gam_v)
  pltpu.sync_copy(beta_hbm, bet_v)

  g = [gam_v[pl.ds(k * 16, 16)] for k in range(4)]
  bt = [bet_v[pl.ds(k * 16, 16)] for k in range(4)]
  lanes = lax.iota(jnp.int32, 16)
  perms = [lanes ^ c for c in (1, 2, 4, 8)]

  def row_body(i, carry):
    cp1 = pltpu.async_copy(
        word_hbm.at[idx_v.at[pl.ds(i * L, 104)]],
        rows_v.at[pl.ds(0, 104)], sem)
    cp2 = pltpu.async_copy(
        word_hbm.at[idx_v.at[pl.ds(i * L + 104, 96)]],
        rows_v.at[pl.ds(104, 96)], sem)
    cp1.wait()
    cp2.wait()

    c = [catrows_v[i, pl.ds(k * 16, 16)] for k in range(4)]

    def tok_body(t, tc):
      e = []
      for k in range(4):
        w = rows_v[t, pl.ds(k * 16, 16)]
        p = pos_v[t, pl.ds(k * 16, 16)]
        e.append(w + p + c[k])
      s = (e[0] + e[1]) + (e[2] + e[3])
      q = (e[0] * e[0] + e[1] * e[1]) + (e[2] * e[2] + e[3] * e[3])
      mu = _lane_allsum(s, perms) * (1.0 / 64.0)
      msq = _lane_allsum(q, perms) * (1.0 / 64.0)
      var = (msq - mu * mu) + EPS
      ii = lax.bitcast_convert_type(var, jnp.int32)
      ii = 0x5F3759DF - lax.shift_right_arithmetic(ii, 1)
      y = lax.bitcast_convert_type(ii, jnp.float32)
      h = var * 0.5
      y = y * (1.5 - h * y * y)
      y = y * (1.5 - h * y * y)
      for k in range(4):
        rows_v[t, pl.ds(k * 16, 16)] = (e[k] - mu) * (g[k] * y) + bt[k]
      return tc

    lax.fori_loop(0, L, tok_body, 0)
    pltpu.sync_copy(rows_v, out_hbm.at[pl.ds(tok0 + i * L, L)])
    return carry

  lax.fori_loop(0, rows_per_w, row_body, 0)


def kernel(word_emb, pos_emb, cat_emb, gamma, beta, tag_tokens, category):
  b, l = tag_tokens.shape
  d = word_emb.shape[1]
  nw = _NC * _NS
  rows_per_w = b // nw
  tag_flat = tag_tokens.reshape(-1).astype(jnp.int32)
  cat_flat = category.reshape(-1).astype(jnp.int32)

  mesh = plsc.VectorSubcoreMesh(core_axis_name="c", subcore_axis_name="s")
  run = pl.kernel(
      functools.partial(_sc_body, l, d, rows_per_w),
      out_type=jax.ShapeDtypeStruct((b * l, d), jnp.float32),
      mesh=mesh,
      compiler_params=pltpu.CompilerParams(use_tc_tiling_on_sc=False),
      scratch_types=[
          pltpu.VMEM((rows_per_w * l,), jnp.int32),
          pltpu.VMEM((l, d), jnp.float32),
          pltpu.VMEM((rows_per_w,), jnp.int32),
          pltpu.VMEM((rows_per_w, d), jnp.float32),
          pltpu.VMEM((d,), jnp.float32),
          pltpu.VMEM((d,), jnp.float32),
          pltpu.VMEM((l, d), jnp.float32),
          pltpu.SemaphoreType.DMA,
      ],
  )
  out = run(word_emb, pos_emb, cat_emb, gamma, beta, tag_flat, cat_flat)
  return out.reshape(b, l, d)


# speedup vs baseline: 1.5435x; 1.5435x over previous
import functools

import jax
import jax.numpy as jnp
from jax import lax
from jax.experimental import pallas as pl
from jax.experimental.pallas import tpu as pltpu
from jax.experimental.pallas import tpu_sc as plsc

EPS = 1e-12
_NC = 2
_NS = 16

_GDN = lax.GatherDimensionNumbers(
    offset_dims=(), collapsed_slice_dims=(0,), start_index_map=(0,))


def _shuffle(x, perm):
  return lax.gather(x, perm[:, None], _GDN, (1,),
                    mode=lax.GatherScatterMode.PROMISE_IN_BOUNDS)


def _lane_allsum(x, perms):
  for p in perms:
    x = x + _shuffle(x, p)
  return x


def _sc_body(L, D, rows_per_w, word_hbm, pos_hbm, cat_hbm, gamma_hbm,
             beta_hbm, tag_hbm, catid_hbm, out_hbm,
             idx_v, pos_v, catid_v, catrows_v, gam_v, bet_v,
             rows0_v, rows1_v, ob0_v, ob1_v,
             gsem0, gsem1, osem0, osem1):
  tok_per_w = rows_per_w * L
  wid = lax.axis_index("c") * _NS + lax.axis_index("s")
  row0 = wid * rows_per_w
  tok0 = wid * tok_per_w
  rows_bufs = (rows0_v, rows1_v)
  out_bufs = (ob0_v, ob1_v)
  gsems = (gsem0, gsem1)
  osems = (osem0, osem1)

  pltpu.sync_copy(tag_hbm.at[pl.ds(tok0, tok_per_w)], idx_v)
  pltpu.sync_copy(pos_hbm.at[pl.ds(0, L)], pos_v)
  pltpu.sync_copy(catid_hbm.at[pl.ds(row0, rows_per_w)], catid_v)
  pltpu.async_copy(cat_hbm.at[catid_v], catrows_v, gsem0).wait()
  pltpu.sync_copy(gamma_hbm, gam_v)
  pltpu.sync_copy(beta_hbm, bet_v)

  g = [gam_v[pl.ds(k * 16, 16)] for k in range(4)]
  bt = [bet_v[pl.ds(k * 16, 16)] for k in range(4)]
  lanes = lax.iota(jnp.int32, 16)
  perms = [lanes ^ c for c in (1, 2, 4, 8)]

  def gather_cps(i, b):
    return (
        pltpu.make_async_copy(
            word_hbm.at[idx_v.at[pl.ds(i * L, 104)]],
            rows_bufs[b].at[pl.ds(0, 104)], gsems[b]),
        pltpu.make_async_copy(
            word_hbm.at[idx_v.at[pl.ds(i * L + 104, 96)]],
            rows_bufs[b].at[pl.ds(104, 96)], gsems[b]),
    )

  def out_cp(i, b):
    return pltpu.make_async_copy(
        out_bufs[b], out_hbm.at[pl.ds(tok0 + i * L, L)], osems[b])

  for cp in gather_cps(0, 0) + gather_cps(1, 1):
    cp.start()

  def pair_body(gg, carry):
    for b in range(2):
      i = 2 * gg + b
      rows_v = rows_bufs[b]
      out_v = out_bufs[b]
      for cp in gather_cps(i, b):
        cp.wait()

      @pl.when(gg >= 1)
      def _():
        out_cp(i - 2, b).wait()

      c = [catrows_v[i, pl.ds(k * 16, 16)] for k in range(4)]

      @plsc.parallel_loop(0, L, 1, unroll=8)
      def _(t):
        e = []
        for k in range(4):
          w = rows_v[t, pl.ds(k * 16, 16)]
          p = pos_v[t, pl.ds(k * 16, 16)]
          e.append(w + p + c[k])
        s = (e[0] + e[1]) + (e[2] + e[3])
        q = (e[0] * e[0] + e[1] * e[1]) + (e[2] * e[2] + e[3] * e[3])
        mu = _lane_allsum(s, perms) * (1.0 / 64.0)
        msq = _lane_allsum(q, perms) * (1.0 / 64.0)
        var = (msq - mu * mu) + EPS
        ii = lax.bitcast_convert_type(var, jnp.int32)
        ii = 0x5F3759DF - lax.shift_right_arithmetic(ii, 1)
        y = lax.bitcast_convert_type(ii, jnp.float32)
        h = var * 0.5
        y = y * (1.5 - h * y * y)
        y = y * (1.5 - h * y * y)
        for k in range(4):
          out_v[t, pl.ds(k * 16, 16)] = (e[k] - mu) * (g[k] * y) + bt[k]

      out_cp(i, b).start()

      @pl.when(gg < rows_per_w // 2 - 1)
      def _():
        for cp in gather_cps(i + 2, b):
          cp.start()
    return carry

  lax.fori_loop(0, rows_per_w // 2, pair_body, 0)

  out_cp(rows_per_w - 2, 0).wait()
  out_cp(rows_per_w - 1, 1).wait()


def kernel(word_emb, pos_emb, cat_emb, gamma, beta, tag_tokens, category):
  b, l = tag_tokens.shape
  d = word_emb.shape[1]
  nw = _NC * _NS
  rows_per_w = b // nw
  tag_flat = tag_tokens.reshape(-1).astype(jnp.int32)
  cat_flat = category.reshape(-1).astype(jnp.int32)

  mesh = plsc.VectorSubcoreMesh(core_axis_name="c", subcore_axis_name="s")
  run = pl.kernel(
      functools.partial(_sc_body, l, d, rows_per_w),
      out_type=jax.ShapeDtypeStruct((b * l, d), jnp.float32),
      mesh=mesh,
      compiler_params=pltpu.CompilerParams(use_tc_tiling_on_sc=False),
      scratch_types=[
          pltpu.VMEM((rows_per_w * l,), jnp.int32),
          pltpu.VMEM((l, d), jnp.float32),
          pltpu.VMEM((rows_per_w,), jnp.int32),
          pltpu.VMEM((rows_per_w, d), jnp.float32),
          pltpu.VMEM((d,), jnp.float32),
          pltpu.VMEM((d,), jnp.float32),
          pltpu.VMEM((l, d), jnp.float32),
          pltpu.VMEM((l, d), jnp.float32),
          pltpu.VMEM((l, d), jnp.float32),
          pltpu.VMEM((l, d), jnp.float32),
          pltpu.SemaphoreType.DMA,
          pltpu.SemaphoreType.DMA,
          pltpu.SemaphoreType.DMA,
          pltpu.SemaphoreType.DMA,
      ],
  )
  out = run(word_emb, pos_emb, cat_emb, gamma, beta, tag_flat, cat_flat)
  return out.reshape(b, l, d)
